# baseline (speedup 1.0000x reference)
import dataclasses

import jax
import jax.numpy as jnp
from jax import lax
from jax.experimental import pallas as pl
from jax.experimental.pallas import tpu as pltpu
from jax.experimental.pallas import tpu_sc as plsc

NN = 10000
NP = 10240
EE = 320000
NC = 2
NS = 16
CHUNK = 128
CPW = 80
EP = NC * NS * CPW * CHUNK
RPT = NP // NS
HR = NP // 16
BLK = 1024


def _sc_mesh():
    return plsc.VectorSubcoreMesh(core_axis_name="c", subcore_axis_name="s")


def _sc_params():
    cp = pltpu.CompilerParams()
    if "needs_layout_passes" in pltpu.CompilerParams.__dataclass_fields__:
        cp = dataclasses.replace(cp, needs_layout_passes=False)
    return cp


def _deg_body(src_hbm, dst_hbm, zin_hbm, io_hbm, outs_hbm, outd_hbm,
              idx_all_s, idx_all_d, iid, iid2, hist, accs_sh, accd_sh,
              zsem, fsem_a, fsem_b):
    c = lax.axis_index("c")
    s = lax.axis_index("s")
    lane = jnp.arange(16, dtype=jnp.int32)
    ones16 = jnp.ones((16,), jnp.float32)

    pltpu.sync_copy(zin_hbm.at[pl.ds(0, 80)], accs_sh.at[pl.ds(s * 80, 80)])
    pltpu.sync_copy(zin_hbm.at[pl.ds(0, 80)], accd_sh.at[pl.ds(s * 80, 80)])
    row_base = (c * NS + s) * CPW
    pltpu.sync_copy(src_hbm.at[pl.ds(row_base, CPW)], idx_all_s)
    pltpu.sync_copy(dst_hbm.at[pl.ds(row_base, CPW)], idx_all_d)
    plsc.subcore_barrier()

    iids = [iid, iid2]
    fsems = [fsem_a, fsem_b]

    def one_hist(idx_all, acc_sel):
        for r in range(2):
            hz = []
            for k in range(HR // CHUNK):
                hz.append(pltpu.async_copy(
                    zin_hbm, hist.at[pl.ds(k * CHUNK, CHUNK)], zsem))
            for h in hz:
                h.wait()

            @pl.loop(0, CPW // 8)
            def _blk(b):
                for jj in range(8):
                    for q in range(8):
                        vec = idx_all[b * 8 + jj, pl.ds(q * 16, 16)]
                        m = vec - (r * 5120)
                        mask = (m >= 0) & (m < 5120)
                        mm = jnp.where(mask, m, 0)
                        vrow = lax.shift_right_logical(mm, 3)
                        vcol = lax.shift_left(lax.bitwise_and(mm, 7), 4) + lane
                        plsc.addupdate_scatter(hist, [vrow, vcol], ones16,
                                               mask=mask)

            fl = [None, None]
            for k in range(HR // CHUNK):
                pk = k & 1
                if fl[pk] is not None:
                    fl[pk].wait()
                pltpu.sync_copy(io_hbm.at[pl.ds(r * HR + k * CHUNK, CHUNK)],
                                iids[pk])
                fl[pk] = pltpu.async_copy(hist.at[pl.ds(k * CHUNK, CHUNK)],
                                          acc_sel.at[iids[pk]], fsems[pk],
                                          add=True)
            fl[0].wait()
            fl[1].wait()

    one_hist(idx_all_s, accs_sh)
    one_hist(idx_all_d, accd_sh)
    plsc.subcore_barrier()

    pltpu.sync_copy(accs_sh.at[pl.ds(s * 80, 80)],
                    outs_hbm.at[pl.ds(c * (2 * HR) + s * 80, 80)])
    pltpu.sync_copy(accd_sh.at[pl.ds(s * 80, 80)],
                    outd_hbm.at[pl.ds(c * (2 * HR) + s * 80, 80)])


@jax.jit
def _deg_call(srcf, dstf, zin, io):
    k = pl.kernel(
        _deg_body,
        out_type=(
            jax.ShapeDtypeStruct((NC * 2 * HR, CHUNK), jnp.float32),
            jax.ShapeDtypeStruct((NC * 2 * HR, CHUNK), jnp.float32),
        ),
        mesh=_sc_mesh(),
        scratch_types=[
            pltpu.VMEM((CPW, CHUNK), jnp.int32),
            pltpu.VMEM((CPW, CHUNK), jnp.int32),
            pltpu.VMEM((CHUNK,), jnp.int32),
            pltpu.VMEM((CHUNK,), jnp.int32),
            pltpu.VMEM((HR, CHUNK), jnp.float32),
            pltpu.VMEM_SHARED((2 * HR, CHUNK), jnp.float32),
            pltpu.VMEM_SHARED((2 * HR, CHUNK), jnp.float32),
            pltpu.SemaphoreType.DMA,
            pltpu.SemaphoreType.DMA,
            pltpu.SemaphoreType.DMA,
        ],
        compiler_params=_sc_params(),
    )
    o1, o2 = k(srcf, dstf, zin, io)
    return (o1.reshape(NC, 2 * HR, CHUNK), o2.reshape(NC, 2 * HR, CHUNK))


def _spmm_body(src_hbm, dst_hbm, tbl_hbm, zin_hbm, out_hbm,
               idx_s8, idx_d8, idx_s8b, idx_d8b, rows_a, rows_b, acc_sh,
               isem_s, isem_d, isem_sb, isem_db,
               gsem_a, gsem_b, gsem_a2, gsem_b2, ssem_a, ssem_b):
    c = lax.axis_index("c")
    s = lax.axis_index("s")

    hz = []
    for k in range(RPT // CHUNK):
        hz.append(pltpu.async_copy(
            zin_hbm, acc_sh.at[pl.ds(s * RPT + k * CHUNK, CHUNK)], isem_s))
    for h in hz:
        h.wait()

    plsc.subcore_barrier()

    row_base = (c * NS + s) * CPW
    bufs = [rows_a, rows_b]
    gsems = [gsem_a, gsem_b]
    gsems2 = [gsem_a2, gsem_b2]
    ssems = [ssem_a, ssem_b]
    ibufs_s = [idx_s8, idx_s8b]
    ibufs_d = [idx_d8, idx_d8b]
    isems_s = [isem_s, isem_sb]
    isems_d = [isem_d, isem_db]

    nblk = CPW // 8
    ih = [None, None]

    def load_idx(b):
        p = b & 1
        r0 = row_base + b * 8
        h1 = pltpu.async_copy(src_hbm.at[pl.ds(r0, 8)], ibufs_s[p], isems_s[p])
        h2 = pltpu.async_copy(dst_hbm.at[pl.ds(r0, 8)], ibufs_d[p], isems_d[p])
        return (h1, h2)

    ih[0] = load_idx(0)
    g = [None, None]
    sct = [None, None]
    for u in range(CPW):
        b, jj = divmod(u, 8)
        p = b & 1
        if jj == 0:
            ih[p][0].wait()
            ih[p][1].wait()
        bi = u & 1
        if sct[bi] is not None:
            sct[bi].wait()
        if jj == 1 and b + 1 < nblk:
            ih[1 - p] = load_idx(b + 1)
        g[bi] = (
            pltpu.async_copy(tbl_hbm.at[ibufs_s[p].at[jj, pl.ds(0, 64)]],
                             bufs[bi].at[pl.ds(0, 64)], gsems[bi]),
            pltpu.async_copy(tbl_hbm.at[ibufs_s[p].at[jj, pl.ds(64, 64)]],
                             bufs[bi].at[pl.ds(64, 64)], gsems2[bi]),
        )
        if u >= 1:
            pb = 1 - bi
            pbb, pjj = divmod(u - 1, 8)
            g[pb][0].wait()
            g[pb][1].wait()
            sct[pb] = pltpu.async_copy(bufs[pb],
                                       acc_sh.at[ibufs_d[pbb & 1].at[pjj]],
                                       ssems[pb], add=True)
    lb = (CPW - 1) & 1
    g[lb][0].wait()
    g[lb][1].wait()
    sct[lb] = pltpu.async_copy(
        bufs[lb], acc_sh.at[ibufs_d[(nblk - 1) & 1].at[7]],
        ssems[lb], add=True)
    sct[0].wait()
    sct[1].wait()

    plsc.subcore_barrier()

    wb = []
    for k in range(RPT // CHUNK):
        r0 = s * RPT + k * CHUNK
        wb.append(pltpu.async_copy(acc_sh.at[pl.ds(r0, CHUNK)],
                                   out_hbm.at[pl.ds(c * NP + r0, CHUNK)],
                                   isem_d))
    for h in wb:
        h.wait()


@jax.jit
def _spmm_call(src2d, dst2d, table, zin):
    k = pl.kernel(
        _spmm_body,
        out_type=jax.ShapeDtypeStruct((NC * NP, CHUNK), jnp.float32),
        mesh=_sc_mesh(),
        scratch_types=[
            pltpu.VMEM((8, CHUNK), jnp.int32),
            pltpu.VMEM((8, CHUNK), jnp.int32),
            pltpu.VMEM((8, CHUNK), jnp.int32),
            pltpu.VMEM((8, CHUNK), jnp.int32),
            pltpu.VMEM((CHUNK, CHUNK), jnp.float32),
            pltpu.VMEM((CHUNK, CHUNK), jnp.float32),
            pltpu.VMEM_SHARED((NP, CHUNK), jnp.float32),
            pltpu.SemaphoreType.DMA,
            pltpu.SemaphoreType.DMA,
            pltpu.SemaphoreType.DMA,
            pltpu.SemaphoreType.DMA,
            pltpu.SemaphoreType.DMA,
            pltpu.SemaphoreType.DMA,
            pltpu.SemaphoreType.DMA,
            pltpu.SemaphoreType.DMA,
            pltpu.SemaphoreType.DMA,
            pltpu.SemaphoreType.DMA,
        ],
    )
    return k(src2d, dst2d, table, zin).reshape(NC, NP, CHUNK)


def _deg_block(dp_ref):
    d = dp_ref[0] + dp_ref[1]
    sel = (lax.broadcasted_iota(jnp.int32, (128, 8), 0) // 16
           == lax.broadcasted_iota(jnp.int32, (128, 8), 1)
           ).astype(jnp.float32)
    return jnp.dot(d, sel, preferred_element_type=jnp.float32)


def _scale_rows(x, n38):
    w = x.shape[1]
    return (x.reshape(128, 8, w) * n38[:, :, None]).reshape(1024, w)


def _norm38(dp_ref):
    return lax.rsqrt(jnp.maximum(_deg_block(dp_ref), 1.0))


def _k1_body(dsp_ref, feat_ref, t1_ref):
    ns = _norm38(dsp_ref)
    t1_ref[...] = _scale_rows(feat_ref[...], ns)


@jax.jit
def _k1_call(dsp, featp):
    return pl.pallas_call(
        _k1_body,
        grid=(NP // BLK,),
        in_specs=[
            pl.BlockSpec((NC, 128, 128), lambda i: (0, i, 0)),
            pl.BlockSpec((BLK, 128), lambda i: (i, 0)),
        ],
        out_specs=pl.BlockSpec((BLK, 128), lambda i: (i, 0)),
        out_shape=jax.ShapeDtypeStruct((NP, 128), jnp.float32),
    )(dsp, featp)


def _k2_body(p_ref, dsp_ref, ddp_ref, w1_ref, b1_ref, w2_ref, t2_ref):
    nd = _norm38(ddp_ref)
    ns = _norm38(dsp_ref)
    aggf = _scale_rows(p_ref[0] + p_ref[1], nd)
    h1 = jnp.maximum(
        jnp.dot(aggf, w1_ref[...], preferred_element_type=jnp.float32)
        + b1_ref[...], 0.0)
    x2 = jnp.dot(_scale_rows(h1, ns), w2_ref[...],
                 preferred_element_type=jnp.float32)
    t2_ref[...] = jnp.concatenate([x2, h1], axis=1)


@jax.jit
def _k2_call(p1, dsp, ddp, W1, b1, W2):
    return pl.pallas_call(
        _k2_body,
        grid=(NP // BLK,),
        in_specs=[
            pl.BlockSpec((NC, BLK, 128), lambda i: (0, i, 0)),
            pl.BlockSpec((NC, 128, 128), lambda i: (0, i, 0)),
            pl.BlockSpec((NC, 128, 128), lambda i: (0, i, 0)),
            pl.BlockSpec((128, 64), lambda i: (0, 0)),
            pl.BlockSpec((1, 64), lambda i: (0, 0)),
            pl.BlockSpec((64, 64), lambda i: (0, 0)),
        ],
        out_specs=pl.BlockSpec((BLK, 128), lambda i: (i, 0)),
        out_shape=jax.ShapeDtypeStruct((NP, 128), jnp.float32),
    )(p1, dsp, ddp, W1, b1, W2)


def _k3_body(p_ref, dsp_ref, ddp_ref, b2_ref, w3_ref, t3_ref):
    nd = _norm38(ddp_ref)
    ns = _norm38(dsp_ref)
    a = p_ref[0] + p_ref[1]
    h2 = jnp.maximum(_scale_rows(a[:, 0:64], nd) + b2_ref[...], 0.0)
    x3 = jnp.dot(_scale_rows(h2, ns), w3_ref[...],
                 preferred_element_type=jnp.float32)
    t3_ref[...] = jnp.concatenate([x3, h2], axis=1)


@jax.jit
def _k3_call(p2, dsp, ddp, b2, W3):
    return pl.pallas_call(
        _k3_body,
        grid=(NP // BLK,),
        in_specs=[
            pl.BlockSpec((NC, BLK, 128), lambda i: (0, i, 0)),
            pl.BlockSpec((NC, 128, 128), lambda i: (0, i, 0)),
            pl.BlockSpec((NC, 128, 128), lambda i: (0, i, 0)),
            pl.BlockSpec((1, 64), lambda i: (0, 0)),
            pl.BlockSpec((64, 64), lambda i: (0, 0)),
        ],
        out_specs=pl.BlockSpec((BLK, 128), lambda i: (i, 0)),
        out_shape=jax.ShapeDtypeStruct((NP, 128), jnp.float32),
    )(p2, dsp, ddp, b2, W3)


def _k4_body(p_ref, ddp_ref, b3_ref, wm_ref, t4_ref):
    nd = _norm38(ddp_ref)
    a = p_ref[0] + p_ref[1]
    h3 = jnp.maximum(_scale_rows(a[:, 0:64], nd) + b3_ref[...], 0.0)
    t4_ref[...] = jnp.dot(h3, wm_ref[128:192, :],
                          preferred_element_type=jnp.float32)


@jax.jit
def _k4_call(p3, ddp, b3, Wm):
    return pl.pallas_call(
        _k4_body,
        grid=(NP // BLK,),
        in_specs=[
            pl.BlockSpec((NC, BLK, 128), lambda i: (0, i, 0)),
            pl.BlockSpec((NC, 128, 128), lambda i: (0, i, 0)),
            pl.BlockSpec((1, 64), lambda i: (0, 0)),
            pl.BlockSpec((192, 128), lambda i: (0, 0)),
        ],
        out_specs=pl.BlockSpec((BLK, 128), lambda i: (i, 0)),
        out_shape=jax.ShapeDtypeStruct((NP, 128), jnp.float32),
    )(p3, ddp, b3, Wm)


def _k5_body(p2_ref, p3_ref, p4_ref, wm_ref, bm_ref, out_ref):
    j1 = p2_ref[0, :, 64:128] + p2_ref[1, :, 64:128]
    j2 = p3_ref[0, :, 64:128] + p3_ref[1, :, 64:128]
    sz3 = p4_ref[0] + p4_ref[1]
    out_ref[...] = (
        jnp.dot(j1, wm_ref[0:64, :], preferred_element_type=jnp.float32)
        + jnp.dot(j2, wm_ref[64:128, :], preferred_element_type=jnp.float32)
        + sz3 + bm_ref[...])


@jax.jit
def _k5_call(p2, p3, p4, Wm, bm):
    return pl.pallas_call(
        _k5_body,
        grid=(NP // BLK,),
        in_specs=[
            pl.BlockSpec((NC, BLK, 128), lambda i: (0, i, 0)),
            pl.BlockSpec((NC, BLK, 128), lambda i: (0, i, 0)),
            pl.BlockSpec((NC, BLK, 128), lambda i: (0, i, 0)),
            pl.BlockSpec((192, 128), lambda i: (0, 0)),
            pl.BlockSpec((1, 128), lambda i: (0, 0)),
        ],
        out_specs=pl.BlockSpec((BLK, 128), lambda i: (i, 0)),
        out_shape=jax.ShapeDtypeStruct((NP, 128), jnp.float32),
    )(p2, p3, p4, Wm, bm)


@jax.jit
def kernel(feat, edge_index, W1, b1, W2, b2, W3, b3, Wm, bm):
    src = edge_index[0]
    dst = edge_index[1]
    pad_n = EP - EE
    pad_idx = NN + (jnp.arange(pad_n, dtype=jnp.int32) % (NP - NN))
    src2d = jnp.concatenate([src, pad_idx]).reshape(EP // CHUNK, CHUNK)
    dst2d = jnp.concatenate([dst, pad_idx]).reshape(EP // CHUNK, CHUNK)
    featp = jnp.pad(feat, ((0, NP - NN), (0, 0)))
    zin = jnp.zeros((CHUNK, CHUNK), jnp.float32)
    io = jnp.arange(2 * HR, dtype=jnp.int32)

    dsp, ddp = _deg_call(src2d, dst2d, zin, io)
    t1 = _k1_call(dsp, featp)
    p1 = _spmm_call(src2d, dst2d, t1, zin)
    t2 = _k2_call(p1, dsp, ddp, W1, b1.reshape(1, 64), W2)
    p2 = _spmm_call(src2d, dst2d, t2, zin)
    t3 = _k3_call(p2, dsp, ddp, b2.reshape(1, 64), W3)
    p3 = _spmm_call(src2d, dst2d, t3, zin)
    t4 = _k4_call(p3, ddp, b3.reshape(1, 64), Wm)
    p4 = _spmm_call(src2d, dst2d, t4, zin)
    out = _k5_call(p2, p3, p4, Wm, bm.reshape(1, 128))
    return out[:NN]


# speedup vs baseline: 1.0108x; 1.0108x over previous
import dataclasses

import jax
import jax.numpy as jnp
from jax import lax
from jax.experimental import pallas as pl
from jax.experimental.pallas import tpu as pltpu
from jax.experimental.pallas import tpu_sc as plsc

NN = 10000
NP = 10240
EE = 320000
NC = 2
NS = 16
CHUNK = 128
CPW = 80
EP = NC * NS * CPW * CHUNK
RPT = NP // NS
HR = NP // 16
BLK = 1024


def _sc_mesh():
    return plsc.VectorSubcoreMesh(core_axis_name="c", subcore_axis_name="s")


def _sc_params():
    cp = pltpu.CompilerParams()
    if "needs_layout_passes" in pltpu.CompilerParams.__dataclass_fields__:
        cp = dataclasses.replace(cp, needs_layout_passes=False)
    return cp


def _deg_body(src_hbm, dst_hbm, zin_hbm, io_hbm, outs_hbm, outd_hbm,
              idx_all_s, idx_all_d, iid, iid2, hist, accs_sh, accd_sh,
              zsem, fsem_a, fsem_b):
    c = lax.axis_index("c")
    s = lax.axis_index("s")
    lane = jnp.arange(16, dtype=jnp.int32)
    ones16 = jnp.ones((16,), jnp.float32)

    pltpu.sync_copy(zin_hbm.at[pl.ds(0, 80)], accs_sh.at[pl.ds(s * 80, 80)])
    pltpu.sync_copy(zin_hbm.at[pl.ds(0, 80)], accd_sh.at[pl.ds(s * 80, 80)])
    row_base = (c * NS + s) * CPW
    pltpu.sync_copy(src_hbm.at[pl.ds(row_base, CPW)], idx_all_s)
    pltpu.sync_copy(dst_hbm.at[pl.ds(row_base, CPW)], idx_all_d)
    plsc.subcore_barrier()

    iids = [iid, iid2]
    fsems = [fsem_a, fsem_b]

    def one_hist(idx_all, acc_sel):
        for r in range(2):
            hz = []
            for k in range(HR // CHUNK):
                hz.append(pltpu.async_copy(
                    zin_hbm, hist.at[pl.ds(k * CHUNK, CHUNK)], zsem))
            for h in hz:
                h.wait()

            @pl.loop(0, CPW // 8)
            def _blk(b):
                for jj in range(8):
                    for q in range(8):
                        vec = idx_all[b * 8 + jj, pl.ds(q * 16, 16)]
                        m = vec - (r * 5120)
                        mask = (m >= 0) & (m < 5120)
                        mm = jnp.where(mask, m, 0)
                        vrow = lax.shift_right_logical(mm, 3)
                        vcol = lax.shift_left(lax.bitwise_and(mm, 7), 4) + lane
                        plsc.addupdate_scatter(hist, [vrow, vcol], ones16,
                                               mask=mask)

            fl = [None, None]
            for k in range(HR // CHUNK):
                pk = k & 1
                if fl[pk] is not None:
                    fl[pk].wait()
                pltpu.sync_copy(io_hbm.at[pl.ds(r * HR + k * CHUNK, CHUNK)],
                                iids[pk])
                fl[pk] = pltpu.async_copy(hist.at[pl.ds(k * CHUNK, CHUNK)],
                                          acc_sel.at[iids[pk]], fsems[pk],
                                          add=True)
            fl[0].wait()
            fl[1].wait()

    one_hist(idx_all_s, accs_sh)
    one_hist(idx_all_d, accd_sh)
    plsc.subcore_barrier()

    pltpu.sync_copy(accs_sh.at[pl.ds(s * 80, 80)],
                    outs_hbm.at[pl.ds(c * (2 * HR) + s * 80, 80)])
    pltpu.sync_copy(accd_sh.at[pl.ds(s * 80, 80)],
                    outd_hbm.at[pl.ds(c * (2 * HR) + s * 80, 80)])


@jax.jit
def _deg_call(srcf, dstf, zin, io):
    k = pl.kernel(
        _deg_body,
        out_type=(
            jax.ShapeDtypeStruct((NC * 2 * HR, CHUNK), jnp.float32),
            jax.ShapeDtypeStruct((NC * 2 * HR, CHUNK), jnp.float32),
        ),
        mesh=_sc_mesh(),
        scratch_types=[
            pltpu.VMEM((CPW, CHUNK), jnp.int32),
            pltpu.VMEM((CPW, CHUNK), jnp.int32),
            pltpu.VMEM((CHUNK,), jnp.int32),
            pltpu.VMEM((CHUNK,), jnp.int32),
            pltpu.VMEM((HR, CHUNK), jnp.float32),
            pltpu.VMEM_SHARED((2 * HR, CHUNK), jnp.float32),
            pltpu.VMEM_SHARED((2 * HR, CHUNK), jnp.float32),
            pltpu.SemaphoreType.DMA,
            pltpu.SemaphoreType.DMA,
            pltpu.SemaphoreType.DMA,
        ],
        compiler_params=_sc_params(),
    )
    o1, o2 = k(srcf, dstf, zin, io)
    return (o1.reshape(NC, 2 * HR, CHUNK), o2.reshape(NC, 2 * HR, CHUNK))


def _spmm_body(src_hbm, dst_hbm, tbl_hbm, zin_hbm, out_hbm,
               idx_s8, idx_d8, idx_s8b, idx_d8b, rows_a, rows_b, acc_sh,
               isem_s, isem_d, isem_sb, isem_db,
               gsem_a, gsem_b, gsem_a2, gsem_b2, ssem_a, ssem_b):
    c = lax.axis_index("c")
    s = lax.axis_index("s")

    hz = []
    for k in range(RPT // CHUNK):
        hz.append(pltpu.async_copy(
            zin_hbm, acc_sh.at[pl.ds(s * RPT + k * CHUNK, CHUNK)], isem_s))
    for h in hz:
        h.wait()

    plsc.subcore_barrier()

    row_base = (c * NS + s) * CPW
    bufs = [rows_a, rows_b]
    gsems = [gsem_a, gsem_b]
    gsems2 = [gsem_a2, gsem_b2]
    ssems = [ssem_a, ssem_b]
    ibufs_s = [idx_s8, idx_s8b]
    ibufs_d = [idx_d8, idx_d8b]
    isems_s = [isem_s, isem_sb]
    isems_d = [isem_d, isem_db]

    nblk = CPW // 8
    ih = [None, None]

    def load_idx(b):
        p = b & 1
        r0 = row_base + b * 8
        h1 = pltpu.async_copy(src_hbm.at[pl.ds(r0, 8)], ibufs_s[p], isems_s[p])
        h2 = pltpu.async_copy(dst_hbm.at[pl.ds(r0, 8)], ibufs_d[p], isems_d[p])
        return (h1, h2)

    ih[0] = load_idx(0)
    g = [None, None]
    sct = [None, None]
    for u in range(CPW):
        b, jj = divmod(u, 8)
        p = b & 1
        if jj == 0:
            ih[p][0].wait()
            ih[p][1].wait()
        bi = u & 1
        if sct[bi] is not None:
            sct[bi].wait()
        if jj == 1 and b + 1 < nblk:
            ih[1 - p] = load_idx(b + 1)
        g[bi] = pltpu.async_copy(tbl_hbm.at[ibufs_s[p].at[jj]], bufs[bi],
                                 gsems[bi])
        if u >= 1:
            pb = 1 - bi
            pbb, pjj = divmod(u - 1, 8)
            g[pb].wait()
            sct[pb] = pltpu.async_copy(bufs[pb],
                                       acc_sh.at[ibufs_d[pbb & 1].at[pjj]],
                                       ssems[pb], add=True)
    lb = (CPW - 1) & 1
    g[lb].wait()
    sct[lb] = pltpu.async_copy(
        bufs[lb], acc_sh.at[ibufs_d[(nblk - 1) & 1].at[7]],
        ssems[lb], add=True)
    sct[0].wait()
    sct[1].wait()

    plsc.subcore_barrier()

    wb = []
    for k in range(RPT // CHUNK):
        r0 = s * RPT + k * CHUNK
        wb.append(pltpu.async_copy(acc_sh.at[pl.ds(r0, CHUNK)],
                                   out_hbm.at[pl.ds(c * NP + r0, CHUNK)],
                                   isem_d))
    for h in wb:
        h.wait()


@jax.jit
def _spmm_call(src2d, dst2d, table, zin):
    k = pl.kernel(
        _spmm_body,
        out_type=jax.ShapeDtypeStruct((NC * NP, CHUNK), jnp.float32),
        mesh=_sc_mesh(),
        scratch_types=[
            pltpu.VMEM((8, CHUNK), jnp.int32),
            pltpu.VMEM((8, CHUNK), jnp.int32),
            pltpu.VMEM((8, CHUNK), jnp.int32),
            pltpu.VMEM((8, CHUNK), jnp.int32),
            pltpu.VMEM((CHUNK, CHUNK), jnp.float32),
            pltpu.VMEM((CHUNK, CHUNK), jnp.float32),
            pltpu.VMEM_SHARED((NP, CHUNK), jnp.float32),
            pltpu.SemaphoreType.DMA,
            pltpu.SemaphoreType.DMA,
            pltpu.SemaphoreType.DMA,
            pltpu.SemaphoreType.DMA,
            pltpu.SemaphoreType.DMA,
            pltpu.SemaphoreType.DMA,
            pltpu.SemaphoreType.DMA,
            pltpu.SemaphoreType.DMA,
            pltpu.SemaphoreType.DMA,
            pltpu.SemaphoreType.DMA,
        ],
    )
    return k(src2d, dst2d, table, zin).reshape(NC, NP, CHUNK)


def _deg_block(dp_ref):
    d = dp_ref[0] + dp_ref[1]
    sel = (lax.broadcasted_iota(jnp.int32, (128, 8), 0) // 16
           == lax.broadcasted_iota(jnp.int32, (128, 8), 1)
           ).astype(jnp.float32)
    return jnp.dot(d, sel, preferred_element_type=jnp.float32)


def _scale_rows(x, n38):
    w = x.shape[1]
    return (x.reshape(128, 8, w) * n38[:, :, None]).reshape(1024, w)


def _norm38(dp_ref):
    return lax.rsqrt(jnp.maximum(_deg_block(dp_ref), 1.0))


def _k1_body(dsp_ref, feat_ref, t1_ref):
    ns = _norm38(dsp_ref)
    t1_ref[...] = _scale_rows(feat_ref[...], ns)


@jax.jit
def _k1_call(dsp, featp):
    return pl.pallas_call(
        _k1_body,
        grid=(NP // BLK,),
        in_specs=[
            pl.BlockSpec((NC, 128, 128), lambda i: (0, i, 0)),
            pl.BlockSpec((BLK, 128), lambda i: (i, 0)),
        ],
        out_specs=pl.BlockSpec((BLK, 128), lambda i: (i, 0)),
        out_shape=jax.ShapeDtypeStruct((NP, 128), jnp.float32),
    )(dsp, featp)


def _k2_body(p_ref, dsp_ref, ddp_ref, w1_ref, b1_ref, w2_ref, t2_ref):
    nd = _norm38(ddp_ref)
    ns = _norm38(dsp_ref)
    aggf = _scale_rows(p_ref[0] + p_ref[1], nd)
    h1 = jnp.maximum(
        jnp.dot(aggf, w1_ref[...], preferred_element_type=jnp.float32)
        + b1_ref[...], 0.0)
    x2 = jnp.dot(_scale_rows(h1, ns), w2_ref[...],
                 preferred_element_type=jnp.float32)
    t2_ref[...] = jnp.concatenate([x2, h1], axis=1)


@jax.jit
def _k2_call(p1, dsp, ddp, W1, b1, W2):
    return pl.pallas_call(
        _k2_body,
        grid=(NP // BLK,),
        in_specs=[
            pl.BlockSpec((NC, BLK, 128), lambda i: (0, i, 0)),
            pl.BlockSpec((NC, 128, 128), lambda i: (0, i, 0)),
            pl.BlockSpec((NC, 128, 128), lambda i: (0, i, 0)),
            pl.BlockSpec((128, 64), lambda i: (0, 0)),
            pl.BlockSpec((1, 64), lambda i: (0, 0)),
            pl.BlockSpec((64, 64), lambda i: (0, 0)),
        ],
        out_specs=pl.BlockSpec((BLK, 128), lambda i: (i, 0)),
        out_shape=jax.ShapeDtypeStruct((NP, 128), jnp.float32),
    )(p1, dsp, ddp, W1, b1, W2)


def _k3_body(p_ref, dsp_ref, ddp_ref, b2_ref, w3_ref, t3_ref):
    nd = _norm38(ddp_ref)
    ns = _norm38(dsp_ref)
    a = p_ref[0] + p_ref[1]
    h2 = jnp.maximum(_scale_rows(a[:, 0:64], nd) + b2_ref[...], 0.0)
    x3 = jnp.dot(_scale_rows(h2, ns), w3_ref[...],
                 preferred_element_type=jnp.float32)
    t3_ref[...] = jnp.concatenate([x3, h2], axis=1)


@jax.jit
def _k3_call(p2, dsp, ddp, b2, W3):
    return pl.pallas_call(
        _k3_body,
        grid=(NP // BLK,),
        in_specs=[
            pl.BlockSpec((NC, BLK, 128), lambda i: (0, i, 0)),
            pl.BlockSpec((NC, 128, 128), lambda i: (0, i, 0)),
            pl.BlockSpec((NC, 128, 128), lambda i: (0, i, 0)),
            pl.BlockSpec((1, 64), lambda i: (0, 0)),
            pl.BlockSpec((64, 64), lambda i: (0, 0)),
        ],
        out_specs=pl.BlockSpec((BLK, 128), lambda i: (i, 0)),
        out_shape=jax.ShapeDtypeStruct((NP, 128), jnp.float32),
    )(p2, dsp, ddp, b2, W3)


def _k4_body(p_ref, ddp_ref, b3_ref, wm_ref, t4_ref):
    nd = _norm38(ddp_ref)
    a = p_ref[0] + p_ref[1]
    h3 = jnp.maximum(_scale_rows(a[:, 0:64], nd) + b3_ref[...], 0.0)
    t4_ref[...] = jnp.dot(h3, wm_ref[128:192, :],
                          preferred_element_type=jnp.float32)


@jax.jit
def _k4_call(p3, ddp, b3, Wm):
    return pl.pallas_call(
        _k4_body,
        grid=(NP // BLK,),
        in_specs=[
            pl.BlockSpec((NC, BLK, 128), lambda i: (0, i, 0)),
            pl.BlockSpec((NC, 128, 128), lambda i: (0, i, 0)),
            pl.BlockSpec((1, 64), lambda i: (0, 0)),
            pl.BlockSpec((192, 128), lambda i: (0, 0)),
        ],
        out_specs=pl.BlockSpec((BLK, 128), lambda i: (i, 0)),
        out_shape=jax.ShapeDtypeStruct((NP, 128), jnp.float32),
    )(p3, ddp, b3, Wm)


def _k5_body(p2_ref, p3_ref, p4_ref, wm_ref, bm_ref, out_ref):
    j1 = p2_ref[0, :, 64:128] + p2_ref[1, :, 64:128]
    j2 = p3_ref[0, :, 64:128] + p3_ref[1, :, 64:128]
    sz3 = p4_ref[0] + p4_ref[1]
    out_ref[...] = (
        jnp.dot(j1, wm_ref[0:64, :], preferred_element_type=jnp.float32)
        + jnp.dot(j2, wm_ref[64:128, :], preferred_element_type=jnp.float32)
        + sz3 + bm_ref[...])


@jax.jit
def _k5_call(p2, p3, p4, Wm, bm):
    return pl.pallas_call(
        _k5_body,
        grid=(NP // BLK,),
        in_specs=[
            pl.BlockSpec((NC, BLK, 128), lambda i: (0, i, 0)),
            pl.BlockSpec((NC, BLK, 128), lambda i: (0, i, 0)),
            pl.BlockSpec((NC, BLK, 128), lambda i: (0, i, 0)),
            pl.BlockSpec((192, 128), lambda i: (0, 0)),
            pl.BlockSpec((1, 128), lambda i: (0, 0)),
        ],
        out_specs=pl.BlockSpec((BLK, 128), lambda i: (i, 0)),
        out_shape=jax.ShapeDtypeStruct((NP, 128), jnp.float32),
    )(p2, p3, p4, Wm, bm)


@jax.jit
def kernel(feat, edge_index, W1, b1, W2, b2, W3, b3, Wm, bm):
    src = edge_index[0]
    dst = edge_index[1]
    pad_n = EP - EE
    pad_idx = NN + (jnp.arange(pad_n, dtype=jnp.int32) % (NP - NN))
    src2d = jnp.concatenate([src, pad_idx]).reshape(EP // CHUNK, CHUNK)
    dst2d = jnp.concatenate([dst, pad_idx]).reshape(EP // CHUNK, CHUNK)
    featp = jnp.pad(feat, ((0, NP - NN), (0, 0)))
    zin = jnp.zeros((CHUNK, CHUNK), jnp.float32)
    io = jnp.arange(2 * HR, dtype=jnp.int32)

    dsp, ddp = _deg_call(src2d, dst2d, zin, io)
    t1 = _k1_call(dsp, featp)
    p1 = _spmm_call(src2d, dst2d, t1, zin)
    t2 = _k2_call(p1, dsp, ddp, W1, b1.reshape(1, 64), W2)
    p2 = _spmm_call(src2d, dst2d, t2, zin)
    t3 = _k3_call(p2, dsp, ddp, b2.reshape(1, 64), W3)
    p3 = _spmm_call(src2d, dst2d, t3, zin)
    t4 = _k4_call(p3, ddp, b3.reshape(1, 64), Wm)
    p4 = _spmm_call(src2d, dst2d, t4, zin)
    out = _k5_call(p2, p3, p4, Wm, bm.reshape(1, 128))
    return out[:NN]
